# baseline (speedup 1.0000x reference)
import functools
import jax
import jax.numpy as jnp
from jax import lax
from jax.experimental import pallas as pl
from jax.experimental.pallas import tpu as pltpu
from jax.experimental.pallas import tpu_sc as plsc

_NC = 2
_NS = 16
_NW = _NC * _NS
_CW = 128
_RPT = 640
_N_PAD = _NS * _RPT


_HALFR = _N_PAD // 2
_ACC_R = 5248
_JUNK = 5200


def _sc_seg_sum(h, src, dst, zeros_blk):
    n, d = h.shape
    e = src.shape[0]
    assert n == _N_PAD and d % _CW == 0
    nchunk = d // _CW
    ew = e // _NS
    K = 80
    nb = ew // K
    assert ew % K == 0 and ew % 16 == 0
    src2 = src.reshape(_NS, ew)
    dst2 = dst.reshape(_NS, ew)
    zero_idx = jnp.zeros((ew + 16,), jnp.int32)
    junk_idx = jnp.full((ew + 16,), _JUNK, jnp.int32)

    mesh = plsc.VectorSubcoreMesh(core_axis_name="c", subcore_axis_name="s")

    @functools.partial(
        pl.kernel, mesh=mesh,
        out_type=jax.ShapeDtypeStruct((n, d), jnp.float32),
        scratch_types=[
            pltpu.VMEM((ew,), jnp.int32),
            pltpu.VMEM((ew,), jnp.int32),
            pltpu.VMEM((ew + 16,), jnp.int32),
            pltpu.VMEM((ew + 16,), jnp.int32),
            pltpu.VMEM((8, K), jnp.int32),
            pltpu.VMEM((K, _CW), jnp.float32),
            pltpu.VMEM((K, _CW), jnp.float32),
            pltpu.VMEM_SHARED((_ACC_R, _CW), jnp.float32),
            pltpu.SemaphoreType.DMA,
            pltpu.SemaphoreType.DMA,
        ],
    )
    def k(h_hbm, src_hbm, dst_hbm, z_hbm, zi_hbm, ji_hbm, out_hbm,
          s_all, d_all, cs, cdf, didxb, rows0, rows1, acc, sem0, sem1):
        cid = lax.axis_index("c")
        sid = lax.axis_index("s")
        lo = cid * _HALFR

        pltpu.sync_copy(src_hbm.at[sid], s_all)
        pltpu.sync_copy(dst_hbm.at[sid], d_all)
        pltpu.sync_copy(zi_hbm, cs)
        pltpu.sync_copy(ji_hbm, cdf)

        lane = lax.iota(jnp.int32, 16)

        def comp(i, off):
            vs = s_all[pl.ds(i * 16, 16)]
            vd = d_all[pl.ds(i * 16, 16)]
            vdr = vd - lo
            m = (vdr >= 0) & (vdr < _HALFR)
            p = jnp.where(m, jnp.int32(1), jnp.int32(0))
            for s in (1, 2, 4, 8):
                sh = jnp.take_along_axis(p, jnp.maximum(lane - s, 0), axis=0)
                p = p + jnp.where(lane >= s, sh, jnp.int32(0))
            srcidx = jnp.zeros((16,), jnp.int32)
            for l in range(16):
                srcidx = srcidx + jnp.where(p[l] <= lane, jnp.int32(1),
                                            jnp.int32(0))
            kk = p[15]
            srcidx = jnp.minimum(srcidx, jnp.int32(15))
            cv_s = jnp.take_along_axis(vs, srcidx, axis=0)
            cv_d = jnp.take_along_axis(vdr, srcidx, axis=0)
            keep = lane < kk
            cs[pl.ds(off, 16)] = jnp.where(keep, cv_s, jnp.int32(0))
            cdf[pl.ds(off, 16)] = jnp.where(keep, cv_d, jnp.int32(_JUNK))
            return off + kk

        cnt = lax.fori_loop(0, ew // 16, comp, jnp.int32(0))
        nbt = jnp.bitwise_or(lax.div(cnt + jnp.int32(K - 1), jnp.int32(K)),
                             jnp.int32(1))

        def chunk_body(ck, _):
            c0 = ck * _CW

            pltpu.sync_copy(z_hbm.at[pl.ds(0, _ACC_R // _NS)],
                            acc.at[pl.ds(sid * (_ACC_R // _NS), _ACC_R // _NS)])
            plsc.subcore_barrier()

            def istart(j, rbuf, sem):
                pltpu.make_async_copy(
                    h_hbm.at[cs.at[pl.ds(j * K, K)], pl.ds(c0, _CW)],
                    rbuf, sem).start()

            def iwait(j, rbuf, sem):
                pltpu.make_async_copy(
                    h_hbm.at[cs.at[pl.ds(j * K, K)], pl.ds(c0, _CW)],
                    rbuf, sem).wait()

            def scat(j, rbuf):
                for l in range(K // 16):
                    didxb[0, pl.ds(l * 16, 16)] = cdf[pl.ds(j * K + l * 16, 16)]
                pltpu.sync_copy(rbuf, acc.at[didxb.at[0]], add=True)

            istart(0, rows0, sem0)

            def body(t, _):
                istart(2 * t + 1, rows1, sem1)
                iwait(2 * t, rows0, sem0)
                scat(2 * t, rows0)
                istart(2 * t + 2, rows0, sem0)
                iwait(2 * t + 1, rows1, sem1)
                scat(2 * t + 1, rows1)
                return 0

            lax.fori_loop(0, lax.div(nbt - 1, jnp.int32(2)), body, 0)
            iwait(nbt - 1, rows0, sem0)
            scat(nbt - 1, rows0)
            plsc.subcore_barrier()

            def wb_body(j, _):
                ra = sid * (_HALFR // _NS) + j * K
                pltpu.sync_copy(acc.at[pl.ds(ra, K)], rows0)
                pltpu.sync_copy(rows0, out_hbm.at[pl.ds(lo + ra, K), pl.ds(c0, _CW)])
                return 0

            lax.fori_loop(0, _HALFR // _NS // K, wb_body, 0)
            plsc.subcore_barrier()
            return 0

        lax.fori_loop(0, nchunk, chunk_body, 0)

    return k(h, src2, dst2, zeros_blk, zero_idx, junk_idx)


def _sc_gather(h, ids):
    n, d = h.shape
    b = ids.shape[0]
    nchunk = d // _CW
    bw = b // _NW
    assert b % _NW == 0 and bw % 8 == 0

    mesh = plsc.VectorSubcoreMesh(core_axis_name="c", subcore_axis_name="s")

    hb = bw // 2
    nw = 2 * nchunk
    assert hb % 8 == 0 and nw % 2 == 0

    @functools.partial(
        pl.kernel, mesh=mesh,
        out_type=jax.ShapeDtypeStruct((b, d), jnp.float32),
        scratch_types=[
            pltpu.VMEM((bw,), jnp.int32),
            pltpu.VMEM((hb, _CW), jnp.float32),
            pltpu.VMEM((hb, _CW), jnp.float32),
            pltpu.SemaphoreType.DMA,
            pltpu.SemaphoreType.DMA,
        ],
    )
    def k(h_hbm, ids_hbm, out_hbm, idx, rows0, rows1, sem0, sem1):
        cid = lax.axis_index("c")
        sid = lax.axis_index("s")
        wid = sid * _NC + cid
        base = wid * bw
        pltpu.sync_copy(ids_hbm.at[pl.ds(base, bw)], idx)

        def cp(w, rbuf, sem):
            c0 = (w // 2) * _CW
            r0 = (w % 2) * hb
            return pltpu.make_async_copy(
                h_hbm.at[idx.at[pl.ds(r0, hb)], pl.ds(c0, _CW)], rbuf, sem)

        def wrb(w, rbuf):
            c0 = (w // 2) * _CW
            r0 = (w % 2) * hb
            pltpu.sync_copy(rbuf, out_hbm.at[pl.ds(base + r0, hb), pl.ds(c0, _CW)])

        cp(0, rows0, sem0).start()

        def w_body(t, _):
            cp(2 * t + 1, rows1, sem1).start()
            cp(2 * t, rows0, sem0).wait()
            wrb(2 * t, rows0)

            @pl.when(2 * t + 2 < nw)
            def _():
                cp(2 * t + 2, rows0, sem0).start()

            cp(2 * t + 1, rows1, sem1).wait()
            wrb(2 * t + 1, rows1)
            return 0

        lax.fori_loop(0, nw // 2, w_body, 0)

    return k(h, ids)


def _bf16_dot(a, w):
    return jnp.dot(a.astype(jnp.bfloat16), w.astype(jnp.bfloat16),
                   preferred_element_type=jnp.float32)


def _layer_body(h_ref, agg_ref, deg_ref, wr_ref, wn_ref, b_ref, out_ref, *, relu):
    h = h_ref[...]
    dinv = 1.0 / jnp.maximum(deg_ref[...], 1.0)
    a = agg_ref[...] * dinv
    acc = _bf16_dot(h, wr_ref[0]) + _bf16_dot(a, wn_ref[0]) + b_ref[0]
    if relu:
        acc = jnp.maximum(acc, 0.0)
    out_ref[...] = acc


def _dual_layer(h, agg, deg, wr, wn, b, *, relu, split_input):
    n = h.shape[0]
    din = wr.shape[1]
    dout = wr.shape[2]
    rb = 1024 if n % 1024 == 0 else n
    nrb = n // rb

    return pl.pallas_call(
        functools.partial(_layer_body, relu=relu),
        grid=(2, nrb),
        in_specs=[
            pl.BlockSpec((rb, din), (lambda g, i: (i, g)) if split_input else (lambda g, i: (i, 0))),
            pl.BlockSpec((rb, din), (lambda g, i: (i, g)) if split_input else (lambda g, i: (i, 0))),
            pl.BlockSpec((rb, 1), lambda g, i: (i, 0)),
            pl.BlockSpec((1, din, dout), lambda g, i: (g, 0, 0)),
            pl.BlockSpec((1, din, dout), lambda g, i: (g, 0, 0)),
            pl.BlockSpec((1, 1, dout), lambda g, i: (g, 0, 0)),
        ],
        out_specs=pl.BlockSpec((rb, dout), lambda g, i: (i, g)),
        out_shape=jax.ShapeDtypeStruct((n, 2 * dout), jnp.float32),
    )(h, agg, deg, wr, wn, b.reshape(2, 1, dout))


def _head_body(sh_ref, th_ref, nh_ref, gt_ref, cm_ref,
               fsw_ref, fsb_ref, ftw_ref, ftb_ref,
               ew1_ref, eb1_ref, ew2_ref, eb2_ref, ew3_ref, eb3_ref,
               nw1_ref, nb1_ref, nw2_ref, nb2_ref,
               pos_ref, neg_ref, pred_ref, loss_ref, acc_ref,
               *, nsteps, btot, h):
    step = pl.program_id(0)

    @pl.when(step == 0)
    def _():
        acc_ref[0] = 0.0
        acc_ref[1] = 0.0
        acc_ref[2] = 0.0

    sh = sh_ref[...]
    th = th_ref[...]
    nh = nh_ref[...]

    src_h = _bf16_dot(sh, fsw_ref[...]) + fsb_ref[0][None, :]
    tgt_h = _bf16_dot(th, ftw_ref[...]) + ftb_ref[0][None, :]
    neg_h = _bf16_dot(nh, ftw_ref[...]) + ftb_ref[0][None, :]

    def link_mlp(z):
        a1 = jnp.maximum(_bf16_dot(z, ew1_ref[...]) + eb1_ref[0][None, :], 0.0)
        a2 = jnp.maximum(_bf16_dot(a1, ew2_ref[...]) + eb2_ref[0][None, :], 0.0)
        return jax.nn.sigmoid(_bf16_dot(a2, ew3_ref[...]) + eb3_ref[0][None, :])

    po = link_mlp(src_h * tgt_h)
    no = link_mlp(src_h * neg_h)
    pos_ref[...] = po
    neg_ref[...] = no

    pn = sh[:, h:]
    p1 = jnp.maximum(_bf16_dot(pn, nw1_ref[...]) + nb1_ref[0][None, :], 0.0)
    pred = _bf16_dot(p1, nw2_ref[...]) + nb2_ref[0][None, :]
    pred_ref[...] = pred

    gt = gt_ref[...]
    cm = cm_ref[...]
    d = pred * cm - gt * cm
    acc_ref[0] += jnp.sum(jnp.log(po + 1e-15))
    acc_ref[1] += jnp.sum(jnp.log(1.0 - no + 1e-15))
    acc_ref[2] += jnp.sum(d * d)

    @pl.when(step == nsteps - 1)
    def _():
        binv = 1.0 / btot
        loss_ref[0, 0] = (-acc_ref[0] * binv) + (-acc_ref[1] * binv) + acc_ref[2] * binv


def _heads(rows, gt, cm, fsw, fsb, ftw, ftb, ep, np_, h):
    b = gt.shape[0]
    g = gt.shape[1]
    rb = 1024 if b % 1024 == 0 else b
    nsteps = b // rb
    (ew1, eb1), (ew2, eb2), (ew3, eb3) = ep
    (nw1, nb1), (nw2, nb2) = np_

    wspec = lambda w: pl.BlockSpec(w.shape, lambda i: (0,) * w.ndim)
    out = pl.pallas_call(
        functools.partial(_head_body, nsteps=nsteps, btot=float(b), h=h),
        grid=(nsteps,),
        in_specs=[
            pl.BlockSpec((rb, 2 * h), lambda i: (i, 0)),
            pl.BlockSpec((rb, 2 * h), lambda i: (i + nsteps, 0)),
            pl.BlockSpec((rb, 2 * h), lambda i: (i + 2 * nsteps, 0)),
            pl.BlockSpec((rb, g), lambda i: (i, 0)),
            pl.BlockSpec((rb, g), lambda i: (i, 0)),
            wspec(fsw), pl.BlockSpec((1, h), lambda i: (0, 0)),
            wspec(ftw), pl.BlockSpec((1, h), lambda i: (0, 0)),
            wspec(ew1), pl.BlockSpec((1, h), lambda i: (0, 0)),
            wspec(ew2), pl.BlockSpec((1, h), lambda i: (0, 0)),
            wspec(ew3), pl.BlockSpec((1, 1), lambda i: (0, 0)),
            wspec(nw1), pl.BlockSpec((1, h), lambda i: (0, 0)),
            wspec(nw2), pl.BlockSpec((1, g), lambda i: (0, 0)),
        ],
        out_specs=[
            pl.BlockSpec((rb, 1), lambda i: (i, 0)),
            pl.BlockSpec((rb, 1), lambda i: (i, 0)),
            pl.BlockSpec((rb, g), lambda i: (i, 0)),
            pl.BlockSpec(memory_space=pltpu.SMEM),
        ],
        out_shape=[
            jax.ShapeDtypeStruct((b, 1), jnp.float32),
            jax.ShapeDtypeStruct((b, 1), jnp.float32),
            jax.ShapeDtypeStruct((b, g), jnp.float32),
            jax.ShapeDtypeStruct((1, 1), jnp.float32),
        ],
        scratch_shapes=[pltpu.SMEM((3,), jnp.float32)],
    )(rows, rows, rows, gt, cm,
      fsw, fsb.reshape(1, -1), ftw, ftb.reshape(1, -1),
      ew1, eb1.reshape(1, -1), ew2, eb2.reshape(1, -1), ew3, eb3.reshape(1, -1),
      nw1, nb1.reshape(1, -1), nw2, nb2.reshape(1, -1))
    pos, neg, pred, loss = out
    return pos, neg, pred, loss[0, 0]


def kernel(x, edge_index, src_ids, tgt_ids, neg_ids, right, num_nodes, gt, cite_mask,
           edge_gnn_params, node_gnn_params, edge_pred_params, node_pred_params,
           fuse_src_W, fuse_src_b, fuse_tgt_W, fuse_tgt_b):
    n, d0 = x.shape
    h = edge_gnn_params[0][0].shape[1]
    src = edge_index[0]
    dst = edge_index[1]

    x_aug = jnp.zeros((_N_PAD, d0 + _CW), jnp.float32)
    x_aug = x_aug.at[:n, :d0].set(x)
    x_aug = x_aug.at[:, d0:].set(1.0)
    zeros_blk = jnp.zeros((_RPT, _CW), jnp.float32)

    def stack(pa, pb, i):
        return (jnp.stack([pa[i][0], pb[i][0]]),
                jnp.stack([pa[i][1], pb[i][1]]),
                jnp.stack([pa[i][2], pb[i][2]]))

    agg0 = _sc_seg_sum(x_aug, src, dst, zeros_blk)
    deg = agg0[:, d0:d0 + 1]
    wr, wn, b = stack(edge_gnn_params, node_gnn_params, 0)
    hc = _dual_layer(x_aug, agg0, deg, wr, wn, b, relu=True, split_input=False)

    for i in (1, 2):
        agg = _sc_seg_sum(hc, src, dst, zeros_blk)
        wr, wn, b = stack(edge_gnn_params, node_gnn_params, i)
        hc = _dual_layer(hc, agg, deg, wr, wn, b, relu=(i < 2), split_input=True)

    ids = jnp.concatenate([src_ids, tgt_ids, neg_ids])
    rows = _sc_gather(hc, ids)

    pos, neg, pred, loss = _heads(
        rows, gt, cite_mask,
        fuse_src_W, fuse_src_b, fuse_tgt_W, fuse_tgt_b,
        edge_pred_params, node_pred_params, h)

    loss = loss + jnp.asarray(right, loss.dtype) * 0.0 + jnp.asarray(num_nodes, loss.dtype) * 0.0
    return (loss, pos, neg, pred)


# speedup vs baseline: 1.0028x; 1.0028x over previous
import functools
import jax
import jax.numpy as jnp
from jax import lax
from jax.experimental import pallas as pl
from jax.experimental.pallas import tpu as pltpu
from jax.experimental.pallas import tpu_sc as plsc

_NC = 2
_NS = 16
_NW = _NC * _NS
_CW = 128
_RPT = 640
_N_PAD = _NS * _RPT


_HALFR = _N_PAD // 2
_ACC_R = 5248
_JUNK = 5200


def _sc_seg_sum(h, src, dst, zeros_blk):
    n, d = h.shape
    e = src.shape[0]
    assert n == _N_PAD and d % _CW == 0
    nchunk = d // _CW
    ew = e // _NS
    K = 80
    nb = ew // K
    assert ew % K == 0 and ew % 16 == 0
    src2 = src.reshape(_NS, ew)
    dst2 = dst.reshape(_NS, ew)
    zero_idx = jnp.zeros((ew + 16,), jnp.int32)
    junk_idx = jnp.full((ew + 16,), _JUNK, jnp.int32)

    mesh = plsc.VectorSubcoreMesh(core_axis_name="c", subcore_axis_name="s")

    @functools.partial(
        pl.kernel, mesh=mesh,
        out_type=jax.ShapeDtypeStruct((n, d), jnp.float32),
        scratch_types=[
            pltpu.VMEM((ew,), jnp.int32),
            pltpu.VMEM((ew,), jnp.int32),
            pltpu.VMEM((ew + 16,), jnp.int32),
            pltpu.VMEM((ew + 16,), jnp.int32),
            pltpu.VMEM((8, K), jnp.int32),
            pltpu.VMEM((K, _CW), jnp.float32),
            pltpu.VMEM((K, _CW), jnp.float32),
            pltpu.VMEM_SHARED((_ACC_R, _CW), jnp.float32),
            pltpu.SemaphoreType.DMA,
            pltpu.SemaphoreType.DMA,
        ],
    )
    def k(h_hbm, src_hbm, dst_hbm, z_hbm, zi_hbm, ji_hbm, out_hbm,
          s_all, d_all, cs, cdf, didxb, rows0, rows1, acc, sem0, sem1):
        cid = lax.axis_index("c")
        sid = lax.axis_index("s")
        lo = cid * _HALFR

        pltpu.sync_copy(src_hbm.at[sid], s_all)
        pltpu.sync_copy(dst_hbm.at[sid], d_all)
        pltpu.sync_copy(zi_hbm, cs)
        pltpu.sync_copy(ji_hbm, cdf)

        lane = lax.iota(jnp.int32, 16)

        def comp(i, off):
            vs = s_all[pl.ds(i * 16, 16)]
            vd = d_all[pl.ds(i * 16, 16)]
            vdr = vd - lo
            m = (vdr >= 0) & (vdr < _HALFR)
            p = jnp.where(m, jnp.int32(1), jnp.int32(0))
            for s in (1, 2, 4, 8):
                sh = jnp.take_along_axis(p, jnp.maximum(lane - s, 0), axis=0)
                p = p + jnp.where(lane >= s, sh, jnp.int32(0))
            srcidx = jnp.zeros((16,), jnp.int32)
            for s in (8, 4, 2, 1):
                t = srcidx + s
                vt = jnp.take_along_axis(p, t - 1, axis=0)
                srcidx = jnp.where(vt <= lane, t, srcidx)
            kk = p[15]
            srcidx = jnp.minimum(srcidx, jnp.int32(15))
            cv_s = jnp.take_along_axis(vs, srcidx, axis=0)
            cv_d = jnp.take_along_axis(vdr, srcidx, axis=0)
            keep = lane < kk
            cs[pl.ds(off, 16)] = jnp.where(keep, cv_s, jnp.int32(0))
            cdf[pl.ds(off, 16)] = jnp.where(keep, cv_d, jnp.int32(_JUNK))
            return off + kk

        cnt = lax.fori_loop(0, ew // 16, comp, jnp.int32(0))
        nbt = jnp.bitwise_or(lax.div(cnt + jnp.int32(K - 1), jnp.int32(K)),
                             jnp.int32(1))

        def chunk_body(ck, _):
            c0 = ck * _CW

            pltpu.sync_copy(z_hbm.at[pl.ds(0, _ACC_R // _NS)],
                            acc.at[pl.ds(sid * (_ACC_R // _NS), _ACC_R // _NS)])
            plsc.subcore_barrier()

            def istart(j, rbuf, sem):
                pltpu.make_async_copy(
                    h_hbm.at[cs.at[pl.ds(j * K, K)], pl.ds(c0, _CW)],
                    rbuf, sem).start()

            def iwait(j, rbuf, sem):
                pltpu.make_async_copy(
                    h_hbm.at[cs.at[pl.ds(j * K, K)], pl.ds(c0, _CW)],
                    rbuf, sem).wait()

            def scat(j, rbuf):
                for l in range(K // 16):
                    didxb[0, pl.ds(l * 16, 16)] = cdf[pl.ds(j * K + l * 16, 16)]
                pltpu.sync_copy(rbuf, acc.at[didxb.at[0]], add=True)

            istart(0, rows0, sem0)

            def body(t, _):
                istart(2 * t + 1, rows1, sem1)
                iwait(2 * t, rows0, sem0)
                scat(2 * t, rows0)
                istart(2 * t + 2, rows0, sem0)
                iwait(2 * t + 1, rows1, sem1)
                scat(2 * t + 1, rows1)
                return 0

            lax.fori_loop(0, lax.div(nbt - 1, jnp.int32(2)), body, 0)
            iwait(nbt - 1, rows0, sem0)
            scat(nbt - 1, rows0)
            plsc.subcore_barrier()

            def wb_body(j, _):
                ra = sid * (_HALFR // _NS) + j * K
                pltpu.sync_copy(acc.at[pl.ds(ra, K)], rows0)
                pltpu.sync_copy(rows0, out_hbm.at[pl.ds(lo + ra, K), pl.ds(c0, _CW)])
                return 0

            lax.fori_loop(0, _HALFR // _NS // K, wb_body, 0)
            plsc.subcore_barrier()
            return 0

        lax.fori_loop(0, nchunk, chunk_body, 0)

    return k(h, src2, dst2, zeros_blk, zero_idx, junk_idx)


def _sc_gather(h, ids):
    n, d = h.shape
    b = ids.shape[0]
    nchunk = d // _CW
    bw = b // _NW
    assert b % _NW == 0 and bw % 8 == 0

    mesh = plsc.VectorSubcoreMesh(core_axis_name="c", subcore_axis_name="s")

    hb = bw // 2
    nw = 2 * nchunk
    assert hb % 8 == 0 and nw % 2 == 0

    @functools.partial(
        pl.kernel, mesh=mesh,
        out_type=jax.ShapeDtypeStruct((b, d), jnp.float32),
        scratch_types=[
            pltpu.VMEM((bw,), jnp.int32),
            pltpu.VMEM((hb, _CW), jnp.float32),
            pltpu.VMEM((hb, _CW), jnp.float32),
            pltpu.SemaphoreType.DMA,
            pltpu.SemaphoreType.DMA,
        ],
    )
    def k(h_hbm, ids_hbm, out_hbm, idx, rows0, rows1, sem0, sem1):
        cid = lax.axis_index("c")
        sid = lax.axis_index("s")
        wid = sid * _NC + cid
        base = wid * bw
        pltpu.sync_copy(ids_hbm.at[pl.ds(base, bw)], idx)

        def cp(w, rbuf, sem):
            c0 = (w // 2) * _CW
            r0 = (w % 2) * hb
            return pltpu.make_async_copy(
                h_hbm.at[idx.at[pl.ds(r0, hb)], pl.ds(c0, _CW)], rbuf, sem)

        def wrb(w, rbuf):
            c0 = (w // 2) * _CW
            r0 = (w % 2) * hb
            pltpu.sync_copy(rbuf, out_hbm.at[pl.ds(base + r0, hb), pl.ds(c0, _CW)])

        cp(0, rows0, sem0).start()

        def w_body(t, _):
            cp(2 * t + 1, rows1, sem1).start()
            cp(2 * t, rows0, sem0).wait()
            wrb(2 * t, rows0)

            @pl.when(2 * t + 2 < nw)
            def _():
                cp(2 * t + 2, rows0, sem0).start()

            cp(2 * t + 1, rows1, sem1).wait()
            wrb(2 * t + 1, rows1)
            return 0

        lax.fori_loop(0, nw // 2, w_body, 0)

    return k(h, ids)


def _bf16_dot(a, w):
    return jnp.dot(a.astype(jnp.bfloat16), w.astype(jnp.bfloat16),
                   preferred_element_type=jnp.float32)


def _layer_body(h_ref, agg_ref, deg_ref, wr_ref, wn_ref, b_ref, out_ref, *, relu):
    h = h_ref[...]
    dinv = 1.0 / jnp.maximum(deg_ref[...], 1.0)
    a = agg_ref[...] * dinv
    acc = _bf16_dot(h, wr_ref[0]) + _bf16_dot(a, wn_ref[0]) + b_ref[0]
    if relu:
        acc = jnp.maximum(acc, 0.0)
    out_ref[...] = acc


def _dual_layer(h, agg, deg, wr, wn, b, *, relu, split_input):
    n = h.shape[0]
    din = wr.shape[1]
    dout = wr.shape[2]
    rb = 1024 if n % 1024 == 0 else n
    nrb = n // rb

    return pl.pallas_call(
        functools.partial(_layer_body, relu=relu),
        grid=(2, nrb),
        in_specs=[
            pl.BlockSpec((rb, din), (lambda g, i: (i, g)) if split_input else (lambda g, i: (i, 0))),
            pl.BlockSpec((rb, din), (lambda g, i: (i, g)) if split_input else (lambda g, i: (i, 0))),
            pl.BlockSpec((rb, 1), lambda g, i: (i, 0)),
            pl.BlockSpec((1, din, dout), lambda g, i: (g, 0, 0)),
            pl.BlockSpec((1, din, dout), lambda g, i: (g, 0, 0)),
            pl.BlockSpec((1, 1, dout), lambda g, i: (g, 0, 0)),
        ],
        out_specs=pl.BlockSpec((rb, dout), lambda g, i: (i, g)),
        out_shape=jax.ShapeDtypeStruct((n, 2 * dout), jnp.float32),
    )(h, agg, deg, wr, wn, b.reshape(2, 1, dout))


def _head_body(sh_ref, th_ref, nh_ref, gt_ref, cm_ref,
               fsw_ref, fsb_ref, ftw_ref, ftb_ref,
               ew1_ref, eb1_ref, ew2_ref, eb2_ref, ew3_ref, eb3_ref,
               nw1_ref, nb1_ref, nw2_ref, nb2_ref,
               pos_ref, neg_ref, pred_ref, loss_ref, acc_ref,
               *, nsteps, btot, h):
    step = pl.program_id(0)

    @pl.when(step == 0)
    def _():
        acc_ref[0] = 0.0
        acc_ref[1] = 0.0
        acc_ref[2] = 0.0

    sh = sh_ref[...]
    th = th_ref[...]
    nh = nh_ref[...]

    src_h = _bf16_dot(sh, fsw_ref[...]) + fsb_ref[0][None, :]
    tgt_h = _bf16_dot(th, ftw_ref[...]) + ftb_ref[0][None, :]
    neg_h = _bf16_dot(nh, ftw_ref[...]) + ftb_ref[0][None, :]

    def link_mlp(z):
        a1 = jnp.maximum(_bf16_dot(z, ew1_ref[...]) + eb1_ref[0][None, :], 0.0)
        a2 = jnp.maximum(_bf16_dot(a1, ew2_ref[...]) + eb2_ref[0][None, :], 0.0)
        return jax.nn.sigmoid(_bf16_dot(a2, ew3_ref[...]) + eb3_ref[0][None, :])

    po = link_mlp(src_h * tgt_h)
    no = link_mlp(src_h * neg_h)
    pos_ref[...] = po
    neg_ref[...] = no

    pn = sh[:, h:]
    p1 = jnp.maximum(_bf16_dot(pn, nw1_ref[...]) + nb1_ref[0][None, :], 0.0)
    pred = _bf16_dot(p1, nw2_ref[...]) + nb2_ref[0][None, :]
    pred_ref[...] = pred

    gt = gt_ref[...]
    cm = cm_ref[...]
    d = pred * cm - gt * cm
    acc_ref[0] += jnp.sum(jnp.log(po + 1e-15))
    acc_ref[1] += jnp.sum(jnp.log(1.0 - no + 1e-15))
    acc_ref[2] += jnp.sum(d * d)

    @pl.when(step == nsteps - 1)
    def _():
        binv = 1.0 / btot
        loss_ref[0, 0] = (-acc_ref[0] * binv) + (-acc_ref[1] * binv) + acc_ref[2] * binv


def _heads(rows, gt, cm, fsw, fsb, ftw, ftb, ep, np_, h):
    b = gt.shape[0]
    g = gt.shape[1]
    rb = 1024 if b % 1024 == 0 else b
    nsteps = b // rb
    (ew1, eb1), (ew2, eb2), (ew3, eb3) = ep
    (nw1, nb1), (nw2, nb2) = np_

    wspec = lambda w: pl.BlockSpec(w.shape, lambda i: (0,) * w.ndim)
    out = pl.pallas_call(
        functools.partial(_head_body, nsteps=nsteps, btot=float(b), h=h),
        grid=(nsteps,),
        in_specs=[
            pl.BlockSpec((rb, 2 * h), lambda i: (i, 0)),
            pl.BlockSpec((rb, 2 * h), lambda i: (i + nsteps, 0)),
            pl.BlockSpec((rb, 2 * h), lambda i: (i + 2 * nsteps, 0)),
            pl.BlockSpec((rb, g), lambda i: (i, 0)),
            pl.BlockSpec((rb, g), lambda i: (i, 0)),
            wspec(fsw), pl.BlockSpec((1, h), lambda i: (0, 0)),
            wspec(ftw), pl.BlockSpec((1, h), lambda i: (0, 0)),
            wspec(ew1), pl.BlockSpec((1, h), lambda i: (0, 0)),
            wspec(ew2), pl.BlockSpec((1, h), lambda i: (0, 0)),
            wspec(ew3), pl.BlockSpec((1, 1), lambda i: (0, 0)),
            wspec(nw1), pl.BlockSpec((1, h), lambda i: (0, 0)),
            wspec(nw2), pl.BlockSpec((1, g), lambda i: (0, 0)),
        ],
        out_specs=[
            pl.BlockSpec((rb, 1), lambda i: (i, 0)),
            pl.BlockSpec((rb, 1), lambda i: (i, 0)),
            pl.BlockSpec((rb, g), lambda i: (i, 0)),
            pl.BlockSpec(memory_space=pltpu.SMEM),
        ],
        out_shape=[
            jax.ShapeDtypeStruct((b, 1), jnp.float32),
            jax.ShapeDtypeStruct((b, 1), jnp.float32),
            jax.ShapeDtypeStruct((b, g), jnp.float32),
            jax.ShapeDtypeStruct((1, 1), jnp.float32),
        ],
        scratch_shapes=[pltpu.SMEM((3,), jnp.float32)],
    )(rows, rows, rows, gt, cm,
      fsw, fsb.reshape(1, -1), ftw, ftb.reshape(1, -1),
      ew1, eb1.reshape(1, -1), ew2, eb2.reshape(1, -1), ew3, eb3.reshape(1, -1),
      nw1, nb1.reshape(1, -1), nw2, nb2.reshape(1, -1))
    pos, neg, pred, loss = out
    return pos, neg, pred, loss[0, 0]


def kernel(x, edge_index, src_ids, tgt_ids, neg_ids, right, num_nodes, gt, cite_mask,
           edge_gnn_params, node_gnn_params, edge_pred_params, node_pred_params,
           fuse_src_W, fuse_src_b, fuse_tgt_W, fuse_tgt_b):
    n, d0 = x.shape
    h = edge_gnn_params[0][0].shape[1]
    src = edge_index[0]
    dst = edge_index[1]

    x_aug = jnp.zeros((_N_PAD, d0 + _CW), jnp.float32)
    x_aug = x_aug.at[:n, :d0].set(x)
    x_aug = x_aug.at[:, d0:].set(1.0)
    zeros_blk = jnp.zeros((_RPT, _CW), jnp.float32)

    def stack(pa, pb, i):
        return (jnp.stack([pa[i][0], pb[i][0]]),
                jnp.stack([pa[i][1], pb[i][1]]),
                jnp.stack([pa[i][2], pb[i][2]]))

    agg0 = _sc_seg_sum(x_aug, src, dst, zeros_blk)
    deg = agg0[:, d0:d0 + 1]
    wr, wn, b = stack(edge_gnn_params, node_gnn_params, 0)
    hc = _dual_layer(x_aug, agg0, deg, wr, wn, b, relu=True, split_input=False)

    for i in (1, 2):
        agg = _sc_seg_sum(hc, src, dst, zeros_blk)
        wr, wn, b = stack(edge_gnn_params, node_gnn_params, i)
        hc = _dual_layer(hc, agg, deg, wr, wn, b, relu=(i < 2), split_input=True)

    ids = jnp.concatenate([src_ids, tgt_ids, neg_ids])
    rows = _sc_gather(hc, ids)

    pos, neg, pred, loss = _heads(
        rows, gt, cite_mask,
        fuse_src_W, fuse_src_b, fuse_tgt_W, fuse_tgt_b,
        edge_pred_params, node_pred_params, h)

    loss = loss + jnp.asarray(right, loss.dtype) * 0.0 + jnp.asarray(num_nodes, loss.dtype) * 0.0
    return (loss, pos, neg, pred)


# speedup vs baseline: 2.1042x; 2.0983x over previous
import functools
import jax
import jax.numpy as jnp
from jax import lax
from jax.experimental import pallas as pl
from jax.experimental.pallas import tpu as pltpu
from jax.experimental.pallas import tpu_sc as plsc

_NC = 2
_NS = 16
_NW = _NC * _NS
_CW = 128
_RPT = 640
_N_PAD = _NS * _RPT


_HALFR = _N_PAD // 2
_ACC_R = 5248
_JUNK = 5200


def _sc_seg_sum(h, src, dst, zeros_blk):
    n, d = h.shape
    e = src.shape[0]
    assert n == _N_PAD and d % _CW == 0
    nchunk = d // _CW
    max_cpc = (nchunk + _NC - 1) // _NC
    ew = e // _NS
    K = 80
    nb = ew // K
    assert ew % K == 0 and nb % 2 == 1 and nb >= 3
    src2 = src.reshape(_NS, ew)
    dst3 = dst.reshape(_NS, nb, K)

    mesh = plsc.VectorSubcoreMesh(core_axis_name="c", subcore_axis_name="s")

    @functools.partial(
        pl.kernel, mesh=mesh,
        out_type=jax.ShapeDtypeStruct((n, d), jnp.float32),
        scratch_types=[
            pltpu.VMEM((ew,), jnp.int32),
            pltpu.VMEM((nb, K), jnp.int32),
            pltpu.VMEM((K, _CW), jnp.float32),
            pltpu.VMEM((K, _CW), jnp.float32),
            pltpu.VMEM_SHARED((n, _CW), jnp.float32),
            pltpu.SemaphoreType.DMA,
            pltpu.SemaphoreType.DMA,
        ],
    )
    def k(h_hbm, src_hbm, dst_hbm, z_hbm, out_hbm,
          sidx, didx, rows0, rows1, acc, sem0, sem1):
        cid = lax.axis_index("c")
        sid = lax.axis_index("s")
        pltpu.sync_copy(src_hbm.at[sid], sidx)
        pltpu.sync_copy(dst_hbm.at[sid], didx)

        def chunk_body(kk, _):
            ck = kk * _NC + cid
            c0 = ck * _CW

            @pl.when(ck < nchunk)
            def _chunk():
                pltpu.sync_copy(z_hbm, acc.at[pl.ds(sid * _RPT, _RPT)])
                plsc.subcore_barrier()

                def istart(j, rbuf, sem):
                    pltpu.make_async_copy(
                        h_hbm.at[sidx.at[pl.ds(j * K, K)], pl.ds(c0, _CW)],
                        rbuf, sem).start()

                def iwait(j, rbuf, sem):
                    pltpu.make_async_copy(
                        h_hbm.at[sidx.at[pl.ds(j * K, K)], pl.ds(c0, _CW)],
                        rbuf, sem).wait()

                def scat(j, rbuf):
                    pltpu.sync_copy(rbuf, acc.at[didx.at[j]], add=True)

                istart(0, rows0, sem0)

                def body(t, _):
                    istart(2 * t + 1, rows1, sem1)
                    iwait(2 * t, rows0, sem0)
                    scat(2 * t, rows0)
                    istart(2 * t + 2, rows0, sem0)
                    iwait(2 * t + 1, rows1, sem1)
                    scat(2 * t + 1, rows1)
                    return 0

                lax.fori_loop(0, (nb - 1) // 2, body, 0)
                iwait(nb - 1, rows0, sem0)
                scat(nb - 1, rows0)

                plsc.subcore_barrier()

                def wb_body(j, _):
                    r0 = sid * _RPT + j * K
                    pltpu.sync_copy(acc.at[pl.ds(r0, K)], rows0)
                    pltpu.sync_copy(rows0, out_hbm.at[pl.ds(r0, K), pl.ds(c0, _CW)])
                    return 0

                lax.fori_loop(0, _RPT // K, wb_body, 0)
                plsc.subcore_barrier()

            return 0

        lax.fori_loop(0, max_cpc, chunk_body, 0)

    return k(h, src2, dst3, zeros_blk)


def _sc_gather(h, ids):
    n, d = h.shape
    b = ids.shape[0]
    nchunk = d // _CW
    bw = b // _NW
    assert b % _NW == 0 and bw % 8 == 0

    mesh = plsc.VectorSubcoreMesh(core_axis_name="c", subcore_axis_name="s")

    hb = bw // 2
    nw = 2 * nchunk
    assert hb % 8 == 0 and nw % 2 == 0

    @functools.partial(
        pl.kernel, mesh=mesh,
        out_type=jax.ShapeDtypeStruct((b, d), jnp.float32),
        scratch_types=[
            pltpu.VMEM((bw,), jnp.int32),
            pltpu.VMEM((hb, _CW), jnp.float32),
            pltpu.VMEM((hb, _CW), jnp.float32),
            pltpu.SemaphoreType.DMA,
            pltpu.SemaphoreType.DMA,
        ],
    )
    def k(h_hbm, ids_hbm, out_hbm, idx, rows0, rows1, sem0, sem1):
        cid = lax.axis_index("c")
        sid = lax.axis_index("s")
        wid = sid * _NC + cid
        base = wid * bw
        pltpu.sync_copy(ids_hbm.at[pl.ds(base, bw)], idx)

        def cp(w, rbuf, sem):
            c0 = (w // 2) * _CW
            r0 = (w % 2) * hb
            return pltpu.make_async_copy(
                h_hbm.at[idx.at[pl.ds(r0, hb)], pl.ds(c0, _CW)], rbuf, sem)

        def wrb(w, rbuf):
            c0 = (w // 2) * _CW
            r0 = (w % 2) * hb
            pltpu.sync_copy(rbuf, out_hbm.at[pl.ds(base + r0, hb), pl.ds(c0, _CW)])

        cp(0, rows0, sem0).start()

        def w_body(t, _):
            cp(2 * t + 1, rows1, sem1).start()
            cp(2 * t, rows0, sem0).wait()
            wrb(2 * t, rows0)

            @pl.when(2 * t + 2 < nw)
            def _():
                cp(2 * t + 2, rows0, sem0).start()

            cp(2 * t + 1, rows1, sem1).wait()
            wrb(2 * t + 1, rows1)
            return 0

        lax.fori_loop(0, nw // 2, w_body, 0)

    return k(h, ids)


def _bf16_dot(a, w):
    return jnp.dot(a.astype(jnp.bfloat16), w.astype(jnp.bfloat16),
                   preferred_element_type=jnp.float32)


def _layer_body(h_ref, agg_ref, deg_ref, wr_ref, wn_ref, b_ref, out_ref, *, relu):
    h = h_ref[...]
    dinv = 1.0 / jnp.maximum(deg_ref[...], 1.0)
    a = agg_ref[...] * dinv
    acc = _bf16_dot(h, wr_ref[0]) + _bf16_dot(a, wn_ref[0]) + b_ref[0]
    if relu:
        acc = jnp.maximum(acc, 0.0)
    out_ref[...] = acc


def _dual_layer(h, agg, deg, wr, wn, b, *, relu, split_input):
    n = h.shape[0]
    din = wr.shape[1]
    dout = wr.shape[2]
    rb = 1024 if n % 1024 == 0 else n
    nrb = n // rb

    return pl.pallas_call(
        functools.partial(_layer_body, relu=relu),
        grid=(2, nrb),
        in_specs=[
            pl.BlockSpec((rb, din), (lambda g, i: (i, g)) if split_input else (lambda g, i: (i, 0))),
            pl.BlockSpec((rb, din), (lambda g, i: (i, g)) if split_input else (lambda g, i: (i, 0))),
            pl.BlockSpec((rb, 1), lambda g, i: (i, 0)),
            pl.BlockSpec((1, din, dout), lambda g, i: (g, 0, 0)),
            pl.BlockSpec((1, din, dout), lambda g, i: (g, 0, 0)),
            pl.BlockSpec((1, 1, dout), lambda g, i: (g, 0, 0)),
        ],
        out_specs=pl.BlockSpec((rb, dout), lambda g, i: (i, g)),
        out_shape=jax.ShapeDtypeStruct((n, 2 * dout), jnp.float32),
    )(h, agg, deg, wr, wn, b.reshape(2, 1, dout))


def _head_body(sh_ref, th_ref, nh_ref, gt_ref, cm_ref,
               fsw_ref, fsb_ref, ftw_ref, ftb_ref,
               ew1_ref, eb1_ref, ew2_ref, eb2_ref, ew3_ref, eb3_ref,
               nw1_ref, nb1_ref, nw2_ref, nb2_ref,
               pos_ref, neg_ref, pred_ref, loss_ref, acc_ref,
               *, nsteps, btot, h):
    step = pl.program_id(0)

    @pl.when(step == 0)
    def _():
        acc_ref[0] = 0.0
        acc_ref[1] = 0.0
        acc_ref[2] = 0.0

    sh = sh_ref[...]
    th = th_ref[...]
    nh = nh_ref[...]

    src_h = _bf16_dot(sh, fsw_ref[...]) + fsb_ref[0][None, :]
    tgt_h = _bf16_dot(th, ftw_ref[...]) + ftb_ref[0][None, :]
    neg_h = _bf16_dot(nh, ftw_ref[...]) + ftb_ref[0][None, :]

    def link_mlp(z):
        a1 = jnp.maximum(_bf16_dot(z, ew1_ref[...]) + eb1_ref[0][None, :], 0.0)
        a2 = jnp.maximum(_bf16_dot(a1, ew2_ref[...]) + eb2_ref[0][None, :], 0.0)
        return jax.nn.sigmoid(_bf16_dot(a2, ew3_ref[...]) + eb3_ref[0][None, :])

    po = link_mlp(src_h * tgt_h)
    no = link_mlp(src_h * neg_h)
    pos_ref[...] = po
    neg_ref[...] = no

    pn = sh[:, h:]
    p1 = jnp.maximum(_bf16_dot(pn, nw1_ref[...]) + nb1_ref[0][None, :], 0.0)
    pred = _bf16_dot(p1, nw2_ref[...]) + nb2_ref[0][None, :]
    pred_ref[...] = pred

    gt = gt_ref[...]
    cm = cm_ref[...]
    d = pred * cm - gt * cm
    acc_ref[0] += jnp.sum(jnp.log(po + 1e-15))
    acc_ref[1] += jnp.sum(jnp.log(1.0 - no + 1e-15))
    acc_ref[2] += jnp.sum(d * d)

    @pl.when(step == nsteps - 1)
    def _():
        binv = 1.0 / btot
        loss_ref[0, 0] = (-acc_ref[0] * binv) + (-acc_ref[1] * binv) + acc_ref[2] * binv


def _heads(rows, gt, cm, fsw, fsb, ftw, ftb, ep, np_, h):
    b = gt.shape[0]
    g = gt.shape[1]
    rb = 1024 if b % 1024 == 0 else b
    nsteps = b // rb
    (ew1, eb1), (ew2, eb2), (ew3, eb3) = ep
    (nw1, nb1), (nw2, nb2) = np_

    wspec = lambda w: pl.BlockSpec(w.shape, lambda i: (0,) * w.ndim)
    out = pl.pallas_call(
        functools.partial(_head_body, nsteps=nsteps, btot=float(b), h=h),
        grid=(nsteps,),
        in_specs=[
            pl.BlockSpec((rb, 2 * h), lambda i: (i, 0)),
            pl.BlockSpec((rb, 2 * h), lambda i: (i + nsteps, 0)),
            pl.BlockSpec((rb, 2 * h), lambda i: (i + 2 * nsteps, 0)),
            pl.BlockSpec((rb, g), lambda i: (i, 0)),
            pl.BlockSpec((rb, g), lambda i: (i, 0)),
            wspec(fsw), pl.BlockSpec((1, h), lambda i: (0, 0)),
            wspec(ftw), pl.BlockSpec((1, h), lambda i: (0, 0)),
            wspec(ew1), pl.BlockSpec((1, h), lambda i: (0, 0)),
            wspec(ew2), pl.BlockSpec((1, h), lambda i: (0, 0)),
            wspec(ew3), pl.BlockSpec((1, 1), lambda i: (0, 0)),
            wspec(nw1), pl.BlockSpec((1, h), lambda i: (0, 0)),
            wspec(nw2), pl.BlockSpec((1, g), lambda i: (0, 0)),
        ],
        out_specs=[
            pl.BlockSpec((rb, 1), lambda i: (i, 0)),
            pl.BlockSpec((rb, 1), lambda i: (i, 0)),
            pl.BlockSpec((rb, g), lambda i: (i, 0)),
            pl.BlockSpec(memory_space=pltpu.SMEM),
        ],
        out_shape=[
            jax.ShapeDtypeStruct((b, 1), jnp.float32),
            jax.ShapeDtypeStruct((b, 1), jnp.float32),
            jax.ShapeDtypeStruct((b, g), jnp.float32),
            jax.ShapeDtypeStruct((1, 1), jnp.float32),
        ],
        scratch_shapes=[pltpu.SMEM((3,), jnp.float32)],
    )(rows, rows, rows, gt, cm,
      fsw, fsb.reshape(1, -1), ftw, ftb.reshape(1, -1),
      ew1, eb1.reshape(1, -1), ew2, eb2.reshape(1, -1), ew3, eb3.reshape(1, -1),
      nw1, nb1.reshape(1, -1), nw2, nb2.reshape(1, -1))
    pos, neg, pred, loss = out
    return pos, neg, pred, loss[0, 0]


def kernel(x, edge_index, src_ids, tgt_ids, neg_ids, right, num_nodes, gt, cite_mask,
           edge_gnn_params, node_gnn_params, edge_pred_params, node_pred_params,
           fuse_src_W, fuse_src_b, fuse_tgt_W, fuse_tgt_b):
    n, d0 = x.shape
    h = edge_gnn_params[0][0].shape[1]
    src = edge_index[0]
    dst = edge_index[1]

    x_aug = jnp.zeros((_N_PAD, d0 + _CW), jnp.float32)
    x_aug = x_aug.at[:n, :d0].set(x)
    x_aug = x_aug.at[:, d0:].set(1.0)
    zeros_blk = jnp.zeros((_RPT, _CW), jnp.float32)

    def stack(pa, pb, i):
        return (jnp.stack([pa[i][0], pb[i][0]]),
                jnp.stack([pa[i][1], pb[i][1]]),
                jnp.stack([pa[i][2], pb[i][2]]))

    agg0 = _sc_seg_sum(x_aug, src, dst, zeros_blk)
    deg = agg0[:, d0:d0 + 1]
    wr, wn, b = stack(edge_gnn_params, node_gnn_params, 0)
    hc = _dual_layer(x_aug, agg0, deg, wr, wn, b, relu=True, split_input=False)

    for i in (1, 2):
        agg = _sc_seg_sum(hc, src, dst, zeros_blk)
        wr, wn, b = stack(edge_gnn_params, node_gnn_params, i)
        hc = _dual_layer(hc, agg, deg, wr, wn, b, relu=(i < 2), split_input=True)

    ids = jnp.concatenate([src_ids, tgt_ids, neg_ids])
    rows = _sc_gather(hc, ids)

    pos, neg, pred, loss = _heads(
        rows, gt, cite_mask,
        fuse_src_W, fuse_src_b, fuse_tgt_W, fuse_tgt_b,
        edge_pred_params, node_pred_params, h)

    loss = loss + jnp.asarray(right, loss.dtype) * 0.0 + jnp.asarray(num_nodes, loss.dtype) * 0.0
    return (loss, pos, neg, pred)


# speedup vs baseline: 2.1352x; 1.0148x over previous
import functools
import jax
import jax.numpy as jnp
from jax import lax
from jax.experimental import pallas as pl
from jax.experimental.pallas import tpu as pltpu
from jax.experimental.pallas import tpu_sc as plsc

_NC = 2
_NS = 16
_NW = _NC * _NS
_CW = 128
_RPT = 640
_N_PAD = _NS * _RPT


_HALFR = _N_PAD // 2
_ACC_R = 5248
_JUNK = 5200


def _sc_seg_sum(h, src, dst, zeros_blk):
    n, d = h.shape
    e = src.shape[0]
    assert n == _N_PAD and d % _CW == 0
    nchunk = d // _CW
    max_cpc = (nchunk + _NC - 1) // _NC
    ew = e // _NS
    K = 80
    nb = ew // K
    assert ew % K == 0 and nb % 2 == 1 and nb >= 3
    src2 = src.reshape(_NS, ew)
    dst3 = dst.reshape(_NS, nb, K)

    mesh = plsc.VectorSubcoreMesh(core_axis_name="c", subcore_axis_name="s")

    @functools.partial(
        pl.kernel, mesh=mesh,
        out_type=jax.ShapeDtypeStruct((n, d), jnp.float32),
        scratch_types=[
            pltpu.VMEM((ew,), jnp.int32),
            pltpu.VMEM((nb, K), jnp.int32),
            pltpu.VMEM((K, _CW), jnp.float32),
            pltpu.VMEM((K, _CW), jnp.float32),
            pltpu.VMEM_SHARED((n, _CW), jnp.float32),
            pltpu.SemaphoreType.DMA,
            pltpu.SemaphoreType.DMA,
        ],
    )
    def k(h_hbm, src_hbm, dst_hbm, z_hbm, out_hbm,
          sidx, didx, rows0, rows1, acc, sem0, sem1):
        cid = lax.axis_index("c")
        sid = lax.axis_index("s")
        pltpu.sync_copy(src_hbm.at[sid], sidx)
        pltpu.sync_copy(dst_hbm.at[sid], didx)

        def chunk_body(kk, _):
            ck = kk * _NC + cid
            c0 = ck * _CW

            @pl.when(ck < nchunk)
            def _chunk():
                pltpu.sync_copy(z_hbm, acc.at[pl.ds(sid * _RPT, _RPT)])
                plsc.subcore_barrier()

                def istart(j, rbuf, sem):
                    pltpu.make_async_copy(
                        h_hbm.at[sidx.at[pl.ds(j * K, K)], pl.ds(c0, _CW)],
                        rbuf, sem).start()

                def iwait(j, rbuf, sem):
                    pltpu.make_async_copy(
                        h_hbm.at[sidx.at[pl.ds(j * K, K)], pl.ds(c0, _CW)],
                        rbuf, sem).wait()

                def scat(j, rbuf):
                    pltpu.sync_copy(rbuf, acc.at[didx.at[j]], add=True)

                istart(0, rows0, sem0)

                def body(t, _):
                    istart(2 * t + 1, rows1, sem1)
                    iwait(2 * t, rows0, sem0)
                    scat(2 * t, rows0)
                    istart(2 * t + 2, rows0, sem0)
                    iwait(2 * t + 1, rows1, sem1)
                    scat(2 * t + 1, rows1)
                    return 0

                lax.fori_loop(0, (nb - 1) // 2, body, 0)
                iwait(nb - 1, rows0, sem0)
                scat(nb - 1, rows0)

                plsc.subcore_barrier()

                def wb_cp(j, rbuf, sem):
                    r0 = sid * _RPT + j * K
                    return pltpu.make_async_copy(
                        rbuf, out_hbm.at[pl.ds(r0, K), pl.ds(c0, _CW)], sem)

                def wb_start(j, rbuf, sem):
                    pltpu.sync_copy(acc.at[pl.ds(sid * _RPT + j * K, K)], rbuf)
                    wb_cp(j, rbuf, sem).start()

                wb_start(0, rows0, sem0)
                wb_start(1, rows1, sem1)

                def wb_body(t, _):
                    wb_cp(2 * t, rows0, sem0).wait()
                    wb_start(2 * t + 2, rows0, sem0)
                    wb_cp(2 * t + 1, rows1, sem1).wait()
                    wb_start(2 * t + 3, rows1, sem1)
                    return 0

                nwb = _RPT // K
                lax.fori_loop(0, nwb // 2 - 1, wb_body, 0)
                wb_cp(nwb - 2, rows0, sem0).wait()
                wb_cp(nwb - 1, rows1, sem1).wait()
                plsc.subcore_barrier()

            return 0

        lax.fori_loop(0, max_cpc, chunk_body, 0)

    return k(h, src2, dst3, zeros_blk)


def _sc_gather(h, ids):
    n, d = h.shape
    b = ids.shape[0]
    nchunk = d // _CW
    bw = b // _NW
    assert b % _NW == 0 and bw % 8 == 0

    mesh = plsc.VectorSubcoreMesh(core_axis_name="c", subcore_axis_name="s")

    hb = bw // 2
    nw = 2 * nchunk
    assert hb % 8 == 0 and nw % 2 == 0

    @functools.partial(
        pl.kernel, mesh=mesh,
        out_type=jax.ShapeDtypeStruct((b, d), jnp.float32),
        scratch_types=[
            pltpu.VMEM((bw,), jnp.int32),
            pltpu.VMEM((hb, _CW), jnp.float32),
            pltpu.VMEM((hb, _CW), jnp.float32),
            pltpu.SemaphoreType.DMA,
            pltpu.SemaphoreType.DMA,
        ],
    )
    def k(h_hbm, ids_hbm, out_hbm, idx, rows0, rows1, sem0, sem1):
        cid = lax.axis_index("c")
        sid = lax.axis_index("s")
        wid = sid * _NC + cid
        base = wid * bw
        pltpu.sync_copy(ids_hbm.at[pl.ds(base, bw)], idx)

        def cp(w, rbuf, sem):
            c0 = (w // 2) * _CW
            r0 = (w % 2) * hb
            return pltpu.make_async_copy(
                h_hbm.at[idx.at[pl.ds(r0, hb)], pl.ds(c0, _CW)], rbuf, sem)

        def wrb(w, rbuf):
            c0 = (w // 2) * _CW
            r0 = (w % 2) * hb
            pltpu.sync_copy(rbuf, out_hbm.at[pl.ds(base + r0, hb), pl.ds(c0, _CW)])

        cp(0, rows0, sem0).start()

        def w_body(t, _):
            cp(2 * t + 1, rows1, sem1).start()
            cp(2 * t, rows0, sem0).wait()
            wrb(2 * t, rows0)

            @pl.when(2 * t + 2 < nw)
            def _():
                cp(2 * t + 2, rows0, sem0).start()

            cp(2 * t + 1, rows1, sem1).wait()
            wrb(2 * t + 1, rows1)
            return 0

        lax.fori_loop(0, nw // 2, w_body, 0)

    return k(h, ids)


def _bf16_dot(a, w):
    return jnp.dot(a.astype(jnp.bfloat16), w.astype(jnp.bfloat16),
                   preferred_element_type=jnp.float32)


def _layer_body(h_ref, agg_ref, deg_ref, wr_ref, wn_ref, b_ref, out_ref, *, relu):
    h = h_ref[...]
    dinv = 1.0 / jnp.maximum(deg_ref[...], 1.0)
    a = agg_ref[...] * dinv
    acc = _bf16_dot(h, wr_ref[0]) + _bf16_dot(a, wn_ref[0]) + b_ref[0]
    if relu:
        acc = jnp.maximum(acc, 0.0)
    out_ref[...] = acc


def _dual_layer(h, agg, deg, wr, wn, b, *, relu, split_input):
    n = h.shape[0]
    din = wr.shape[1]
    dout = wr.shape[2]
    rb = 1024 if n % 1024 == 0 else n
    nrb = n // rb

    return pl.pallas_call(
        functools.partial(_layer_body, relu=relu),
        grid=(2, nrb),
        in_specs=[
            pl.BlockSpec((rb, din), (lambda g, i: (i, g)) if split_input else (lambda g, i: (i, 0))),
            pl.BlockSpec((rb, din), (lambda g, i: (i, g)) if split_input else (lambda g, i: (i, 0))),
            pl.BlockSpec((rb, 1), lambda g, i: (i, 0)),
            pl.BlockSpec((1, din, dout), lambda g, i: (g, 0, 0)),
            pl.BlockSpec((1, din, dout), lambda g, i: (g, 0, 0)),
            pl.BlockSpec((1, 1, dout), lambda g, i: (g, 0, 0)),
        ],
        out_specs=pl.BlockSpec((rb, dout), lambda g, i: (i, g)),
        out_shape=jax.ShapeDtypeStruct((n, 2 * dout), jnp.float32),
    )(h, agg, deg, wr, wn, b.reshape(2, 1, dout))


def _head_body(sh_ref, th_ref, nh_ref, gt_ref, cm_ref,
               fsw_ref, fsb_ref, ftw_ref, ftb_ref,
               ew1_ref, eb1_ref, ew2_ref, eb2_ref, ew3_ref, eb3_ref,
               nw1_ref, nb1_ref, nw2_ref, nb2_ref,
               pos_ref, neg_ref, pred_ref, loss_ref, acc_ref,
               *, nsteps, btot, h):
    step = pl.program_id(0)

    @pl.when(step == 0)
    def _():
        acc_ref[0] = 0.0
        acc_ref[1] = 0.0
        acc_ref[2] = 0.0

    sh = sh_ref[...]
    th = th_ref[...]
    nh = nh_ref[...]

    src_h = _bf16_dot(sh, fsw_ref[...]) + fsb_ref[0][None, :]
    tgt_h = _bf16_dot(th, ftw_ref[...]) + ftb_ref[0][None, :]
    neg_h = _bf16_dot(nh, ftw_ref[...]) + ftb_ref[0][None, :]

    def link_mlp(z):
        a1 = jnp.maximum(_bf16_dot(z, ew1_ref[...]) + eb1_ref[0][None, :], 0.0)
        a2 = jnp.maximum(_bf16_dot(a1, ew2_ref[...]) + eb2_ref[0][None, :], 0.0)
        return jax.nn.sigmoid(_bf16_dot(a2, ew3_ref[...]) + eb3_ref[0][None, :])

    po = link_mlp(src_h * tgt_h)
    no = link_mlp(src_h * neg_h)
    pos_ref[...] = po
    neg_ref[...] = no

    pn = sh[:, h:]
    p1 = jnp.maximum(_bf16_dot(pn, nw1_ref[...]) + nb1_ref[0][None, :], 0.0)
    pred = _bf16_dot(p1, nw2_ref[...]) + nb2_ref[0][None, :]
    pred_ref[...] = pred

    gt = gt_ref[...]
    cm = cm_ref[...]
    d = pred * cm - gt * cm
    acc_ref[0] += jnp.sum(jnp.log(po + 1e-15))
    acc_ref[1] += jnp.sum(jnp.log(1.0 - no + 1e-15))
    acc_ref[2] += jnp.sum(d * d)

    @pl.when(step == nsteps - 1)
    def _():
        binv = 1.0 / btot
        loss_ref[0, 0] = (-acc_ref[0] * binv) + (-acc_ref[1] * binv) + acc_ref[2] * binv


def _heads(rows, gt, cm, fsw, fsb, ftw, ftb, ep, np_, h):
    b = gt.shape[0]
    g = gt.shape[1]
    rb = 1024 if b % 1024 == 0 else b
    nsteps = b // rb
    (ew1, eb1), (ew2, eb2), (ew3, eb3) = ep
    (nw1, nb1), (nw2, nb2) = np_

    wspec = lambda w: pl.BlockSpec(w.shape, lambda i: (0,) * w.ndim)
    out = pl.pallas_call(
        functools.partial(_head_body, nsteps=nsteps, btot=float(b), h=h),
        grid=(nsteps,),
        in_specs=[
            pl.BlockSpec((rb, 2 * h), lambda i: (i, 0)),
            pl.BlockSpec((rb, 2 * h), lambda i: (i + nsteps, 0)),
            pl.BlockSpec((rb, 2 * h), lambda i: (i + 2 * nsteps, 0)),
            pl.BlockSpec((rb, g), lambda i: (i, 0)),
            pl.BlockSpec((rb, g), lambda i: (i, 0)),
            wspec(fsw), pl.BlockSpec((1, h), lambda i: (0, 0)),
            wspec(ftw), pl.BlockSpec((1, h), lambda i: (0, 0)),
            wspec(ew1), pl.BlockSpec((1, h), lambda i: (0, 0)),
            wspec(ew2), pl.BlockSpec((1, h), lambda i: (0, 0)),
            wspec(ew3), pl.BlockSpec((1, 1), lambda i: (0, 0)),
            wspec(nw1), pl.BlockSpec((1, h), lambda i: (0, 0)),
            wspec(nw2), pl.BlockSpec((1, g), lambda i: (0, 0)),
        ],
        out_specs=[
            pl.BlockSpec((rb, 1), lambda i: (i, 0)),
            pl.BlockSpec((rb, 1), lambda i: (i, 0)),
            pl.BlockSpec((rb, g), lambda i: (i, 0)),
            pl.BlockSpec(memory_space=pltpu.SMEM),
        ],
        out_shape=[
            jax.ShapeDtypeStruct((b, 1), jnp.float32),
            jax.ShapeDtypeStruct((b, 1), jnp.float32),
            jax.ShapeDtypeStruct((b, g), jnp.float32),
            jax.ShapeDtypeStruct((1, 1), jnp.float32),
        ],
        scratch_shapes=[pltpu.SMEM((3,), jnp.float32)],
    )(rows, rows, rows, gt, cm,
      fsw, fsb.reshape(1, -1), ftw, ftb.reshape(1, -1),
      ew1, eb1.reshape(1, -1), ew2, eb2.reshape(1, -1), ew3, eb3.reshape(1, -1),
      nw1, nb1.reshape(1, -1), nw2, nb2.reshape(1, -1))
    pos, neg, pred, loss = out
    return pos, neg, pred, loss[0, 0]


def kernel(x, edge_index, src_ids, tgt_ids, neg_ids, right, num_nodes, gt, cite_mask,
           edge_gnn_params, node_gnn_params, edge_pred_params, node_pred_params,
           fuse_src_W, fuse_src_b, fuse_tgt_W, fuse_tgt_b):
    n, d0 = x.shape
    h = edge_gnn_params[0][0].shape[1]
    src = edge_index[0]
    dst = edge_index[1]

    x_aug = jnp.zeros((_N_PAD, d0 + _CW), jnp.float32)
    x_aug = x_aug.at[:n, :d0].set(x)
    x_aug = x_aug.at[:, d0:].set(1.0)
    zeros_blk = jnp.zeros((_RPT, _CW), jnp.float32)

    def stack(pa, pb, i):
        return (jnp.stack([pa[i][0], pb[i][0]]),
                jnp.stack([pa[i][1], pb[i][1]]),
                jnp.stack([pa[i][2], pb[i][2]]))

    agg0 = _sc_seg_sum(x_aug, src, dst, zeros_blk)
    deg = agg0[:, d0:d0 + 1]
    wr, wn, b = stack(edge_gnn_params, node_gnn_params, 0)
    hc = _dual_layer(x_aug, agg0, deg, wr, wn, b, relu=True, split_input=False)

    for i in (1, 2):
        agg = _sc_seg_sum(hc, src, dst, zeros_blk)
        wr, wn, b = stack(edge_gnn_params, node_gnn_params, i)
        hc = _dual_layer(hc, agg, deg, wr, wn, b, relu=(i < 2), split_input=True)

    ids = jnp.concatenate([src_ids, tgt_ids, neg_ids])
    rows = _sc_gather(hc, ids)

    pos, neg, pred, loss = _heads(
        rows, gt, cite_mask,
        fuse_src_W, fuse_src_b, fuse_tgt_W, fuse_tgt_b,
        edge_pred_params, node_pred_params, h)

    loss = loss + jnp.asarray(right, loss.dtype) * 0.0 + jnp.asarray(num_nodes, loss.dtype) * 0.0
    return (loss, pos, neg, pred)


# speedup vs baseline: 2.1553x; 1.0094x over previous
import functools
import jax
import jax.numpy as jnp
from jax import lax
from jax.experimental import pallas as pl
from jax.experimental.pallas import tpu as pltpu
from jax.experimental.pallas import tpu_sc as plsc

_NC = 2
_NS = 16
_NW = _NC * _NS
_CW = 128
_RPT = 640
_N_PAD = _NS * _RPT


_HALFR = _N_PAD // 2
_ACC_R = 5248
_JUNK = 5200


def _sc_seg_sum(h, src, dst, zeros_blk):
    n, d = h.shape
    e = src.shape[0]
    assert n == _N_PAD and d % _CW == 0
    nchunk = d // _CW
    max_cpc = (nchunk + _NC - 1) // _NC
    ew = e // _NS
    K = 80
    nb = ew // K
    assert ew % K == 0 and nb % 2 == 1 and nb >= 3
    src2 = src.reshape(_NS, ew)
    dst3 = dst.reshape(_NS, nb, K)

    mesh = plsc.VectorSubcoreMesh(core_axis_name="c", subcore_axis_name="s")

    @functools.partial(
        pl.kernel, mesh=mesh,
        out_type=jax.ShapeDtypeStruct((n, d), jnp.float32),
        scratch_types=[
            pltpu.VMEM((ew,), jnp.int32),
            pltpu.VMEM((nb, K), jnp.int32),
            pltpu.VMEM((K, _CW), jnp.float32),
            pltpu.VMEM((K, _CW), jnp.float32),
            pltpu.VMEM_SHARED((n, _CW), jnp.float32),
            pltpu.SemaphoreType.DMA,
            pltpu.SemaphoreType.DMA,
        ],
    )
    def k(h_hbm, src_hbm, dst_hbm, z_hbm, out_hbm,
          sidx, didx, rows0, rows1, acc, sem0, sem1):
        cid = lax.axis_index("c")
        sid = lax.axis_index("s")
        pltpu.sync_copy(src_hbm.at[sid], sidx)
        pltpu.sync_copy(dst_hbm.at[sid], didx)

        def chunk_body(kk, _):
            ck = kk * _NC + cid
            c0 = ck * _CW

            @pl.when(ck < nchunk)
            def _chunk():
                pltpu.sync_copy(z_hbm, acc.at[pl.ds(sid * _RPT, _RPT)])
                plsc.subcore_barrier()

                def istart(j, rbuf, sem):
                    pltpu.make_async_copy(
                        h_hbm.at[sidx.at[pl.ds(j * K, K)], pl.ds(c0, _CW)],
                        rbuf, sem).start()

                def iwait(j, rbuf, sem):
                    pltpu.make_async_copy(
                        h_hbm.at[sidx.at[pl.ds(j * K, K)], pl.ds(c0, _CW)],
                        rbuf, sem).wait()

                def scat(j, rbuf):
                    pltpu.sync_copy(rbuf, acc.at[didx.at[j]], add=True)

                istart(0, rows0, sem0)

                def body(t, _):
                    istart(2 * t + 1, rows1, sem1)
                    iwait(2 * t, rows0, sem0)
                    scat(2 * t, rows0)
                    istart(2 * t + 2, rows0, sem0)
                    iwait(2 * t + 1, rows1, sem1)
                    scat(2 * t + 1, rows1)
                    return 0

                lax.fori_loop(0, (nb - 1) // 2, body, 0)
                iwait(nb - 1, rows0, sem0)
                scat(nb - 1, rows0)

                plsc.subcore_barrier()

                def wb_cp(j, rbuf, sem):
                    r0 = sid * _RPT + j * K
                    return pltpu.make_async_copy(
                        rbuf, out_hbm.at[pl.ds(r0, K), pl.ds(c0, _CW)], sem)

                def wb_start(j, rbuf, sem):
                    pltpu.sync_copy(acc.at[pl.ds(sid * _RPT + j * K, K)], rbuf)
                    wb_cp(j, rbuf, sem).start()

                wb_start(0, rows0, sem0)
                wb_start(1, rows1, sem1)

                def wb_body(t, _):
                    wb_cp(2 * t, rows0, sem0).wait()
                    wb_start(2 * t + 2, rows0, sem0)
                    wb_cp(2 * t + 1, rows1, sem1).wait()
                    wb_start(2 * t + 3, rows1, sem1)
                    return 0

                nwb = _RPT // K
                lax.fori_loop(0, nwb // 2 - 1, wb_body, 0)
                wb_cp(nwb - 2, rows0, sem0).wait()
                wb_cp(nwb - 1, rows1, sem1).wait()
                plsc.subcore_barrier()

            return 0

        lax.fori_loop(0, max_cpc, chunk_body, 0)

    return k(h, src2, dst3, zeros_blk)


def _sc_gather(h, ids):
    n, d = h.shape
    b = ids.shape[0]
    nchunk = d // _CW
    bw = b // _NW
    assert b % _NW == 0 and bw % 8 == 0

    mesh = plsc.VectorSubcoreMesh(core_axis_name="c", subcore_axis_name="s")

    hb = bw // 2
    nw = 2 * nchunk
    assert hb % 8 == 0 and nw % 2 == 0

    @functools.partial(
        pl.kernel, mesh=mesh,
        out_type=jax.ShapeDtypeStruct((b, d), jnp.float32),
        scratch_types=[
            pltpu.VMEM((bw,), jnp.int32),
            pltpu.VMEM((hb, _CW), jnp.float32),
            pltpu.VMEM((hb, _CW), jnp.float32),
            pltpu.SemaphoreType.DMA,
            pltpu.SemaphoreType.DMA,
        ],
    )
    def k(h_hbm, ids_hbm, out_hbm, idx, rows0, rows1, sem0, sem1):
        cid = lax.axis_index("c")
        sid = lax.axis_index("s")
        wid = sid * _NC + cid
        base = wid * bw
        pltpu.sync_copy(ids_hbm.at[pl.ds(base, bw)], idx)

        def cp(w, rbuf, sem):
            c0 = (w // 2) * _CW
            r0 = (w % 2) * hb
            return pltpu.make_async_copy(
                h_hbm.at[idx.at[pl.ds(r0, hb)], pl.ds(c0, _CW)], rbuf, sem)

        def wrb(w, rbuf):
            c0 = (w // 2) * _CW
            r0 = (w % 2) * hb
            pltpu.sync_copy(rbuf, out_hbm.at[pl.ds(base + r0, hb), pl.ds(c0, _CW)])

        cp(0, rows0, sem0).start()

        def w_body(t, _):
            cp(2 * t + 1, rows1, sem1).start()
            cp(2 * t, rows0, sem0).wait()
            wrb(2 * t, rows0)

            @pl.when(2 * t + 2 < nw)
            def _():
                cp(2 * t + 2, rows0, sem0).start()

            cp(2 * t + 1, rows1, sem1).wait()
            wrb(2 * t + 1, rows1)
            return 0

        lax.fori_loop(0, nw // 2, w_body, 0)

    return k(h, ids)


def _bf16_dot(a, w):
    return jnp.dot(a.astype(jnp.bfloat16), w.astype(jnp.bfloat16),
                   preferred_element_type=jnp.float32)


def _layer_body(h_ref, agg_ref, deg_ref, wr_ref, wn_ref, b_ref, out_ref, *, relu):
    h = h_ref[...]
    dinv = 1.0 / jnp.maximum(deg_ref[...], 1.0)
    a = agg_ref[...] * dinv
    acc = _bf16_dot(h, wr_ref[0]) + _bf16_dot(a, wn_ref[0]) + b_ref[0]
    if relu:
        acc = jnp.maximum(acc, 0.0)
    out_ref[...] = acc


def _dual_layer(h, agg, deg, wr, wn, b, *, relu, split_input):
    n = h.shape[0]
    din = wr.shape[1]
    dout = wr.shape[2]
    rb = 2048 if n % 2048 == 0 else n
    nrb = n // rb

    return pl.pallas_call(
        functools.partial(_layer_body, relu=relu),
        grid=(2, nrb),
        in_specs=[
            pl.BlockSpec((rb, din), (lambda g, i: (i, g)) if split_input else (lambda g, i: (i, 0))),
            pl.BlockSpec((rb, din), (lambda g, i: (i, g)) if split_input else (lambda g, i: (i, 0))),
            pl.BlockSpec((rb, 1), lambda g, i: (i, 0)),
            pl.BlockSpec((1, din, dout), lambda g, i: (g, 0, 0)),
            pl.BlockSpec((1, din, dout), lambda g, i: (g, 0, 0)),
            pl.BlockSpec((1, 1, dout), lambda g, i: (g, 0, 0)),
        ],
        out_specs=pl.BlockSpec((rb, dout), lambda g, i: (i, g)),
        out_shape=jax.ShapeDtypeStruct((n, 2 * dout), jnp.float32),
    )(h, agg, deg, wr, wn, b.reshape(2, 1, dout))


def _head_body(sh_ref, th_ref, nh_ref, gt_ref, cm_ref,
               fsw_ref, fsb_ref, ftw_ref, ftb_ref,
               ew1_ref, eb1_ref, ew2_ref, eb2_ref, ew3_ref, eb3_ref,
               nw1_ref, nb1_ref, nw2_ref, nb2_ref,
               pos_ref, neg_ref, pred_ref, loss_ref, acc_ref,
               *, nsteps, btot, h):
    step = pl.program_id(0)

    @pl.when(step == 0)
    def _():
        acc_ref[0] = 0.0
        acc_ref[1] = 0.0
        acc_ref[2] = 0.0

    sh = sh_ref[...]
    th = th_ref[...]
    nh = nh_ref[...]

    src_h = _bf16_dot(sh, fsw_ref[...]) + fsb_ref[0][None, :]
    tgt_h = _bf16_dot(th, ftw_ref[...]) + ftb_ref[0][None, :]
    neg_h = _bf16_dot(nh, ftw_ref[...]) + ftb_ref[0][None, :]

    def link_mlp(z):
        a1 = jnp.maximum(_bf16_dot(z, ew1_ref[...]) + eb1_ref[0][None, :], 0.0)
        a2 = jnp.maximum(_bf16_dot(a1, ew2_ref[...]) + eb2_ref[0][None, :], 0.0)
        return jax.nn.sigmoid(_bf16_dot(a2, ew3_ref[...]) + eb3_ref[0][None, :])

    po = link_mlp(src_h * tgt_h)
    no = link_mlp(src_h * neg_h)
    pos_ref[...] = po
    neg_ref[...] = no

    pn = sh[:, h:]
    p1 = jnp.maximum(_bf16_dot(pn, nw1_ref[...]) + nb1_ref[0][None, :], 0.0)
    pred = _bf16_dot(p1, nw2_ref[...]) + nb2_ref[0][None, :]
    pred_ref[...] = pred

    gt = gt_ref[...]
    cm = cm_ref[...]
    d = pred * cm - gt * cm
    acc_ref[0] += jnp.sum(jnp.log(po + 1e-15))
    acc_ref[1] += jnp.sum(jnp.log(1.0 - no + 1e-15))
    acc_ref[2] += jnp.sum(d * d)

    @pl.when(step == nsteps - 1)
    def _():
        binv = 1.0 / btot
        loss_ref[0, 0] = (-acc_ref[0] * binv) + (-acc_ref[1] * binv) + acc_ref[2] * binv


def _heads(rows, gt, cm, fsw, fsb, ftw, ftb, ep, np_, h):
    b = gt.shape[0]
    g = gt.shape[1]
    rb = 1024 if b % 1024 == 0 else b
    nsteps = b // rb
    (ew1, eb1), (ew2, eb2), (ew3, eb3) = ep
    (nw1, nb1), (nw2, nb2) = np_

    wspec = lambda w: pl.BlockSpec(w.shape, lambda i: (0,) * w.ndim)
    out = pl.pallas_call(
        functools.partial(_head_body, nsteps=nsteps, btot=float(b), h=h),
        grid=(nsteps,),
        in_specs=[
            pl.BlockSpec((rb, 2 * h), lambda i: (i, 0)),
            pl.BlockSpec((rb, 2 * h), lambda i: (i + nsteps, 0)),
            pl.BlockSpec((rb, 2 * h), lambda i: (i + 2 * nsteps, 0)),
            pl.BlockSpec((rb, g), lambda i: (i, 0)),
            pl.BlockSpec((rb, g), lambda i: (i, 0)),
            wspec(fsw), pl.BlockSpec((1, h), lambda i: (0, 0)),
            wspec(ftw), pl.BlockSpec((1, h), lambda i: (0, 0)),
            wspec(ew1), pl.BlockSpec((1, h), lambda i: (0, 0)),
            wspec(ew2), pl.BlockSpec((1, h), lambda i: (0, 0)),
            wspec(ew3), pl.BlockSpec((1, 1), lambda i: (0, 0)),
            wspec(nw1), pl.BlockSpec((1, h), lambda i: (0, 0)),
            wspec(nw2), pl.BlockSpec((1, g), lambda i: (0, 0)),
        ],
        out_specs=[
            pl.BlockSpec((rb, 1), lambda i: (i, 0)),
            pl.BlockSpec((rb, 1), lambda i: (i, 0)),
            pl.BlockSpec((rb, g), lambda i: (i, 0)),
            pl.BlockSpec(memory_space=pltpu.SMEM),
        ],
        out_shape=[
            jax.ShapeDtypeStruct((b, 1), jnp.float32),
            jax.ShapeDtypeStruct((b, 1), jnp.float32),
            jax.ShapeDtypeStruct((b, g), jnp.float32),
            jax.ShapeDtypeStruct((1, 1), jnp.float32),
        ],
        scratch_shapes=[pltpu.SMEM((3,), jnp.float32)],
    )(rows, rows, rows, gt, cm,
      fsw, fsb.reshape(1, -1), ftw, ftb.reshape(1, -1),
      ew1, eb1.reshape(1, -1), ew2, eb2.reshape(1, -1), ew3, eb3.reshape(1, -1),
      nw1, nb1.reshape(1, -1), nw2, nb2.reshape(1, -1))
    pos, neg, pred, loss = out
    return pos, neg, pred, loss[0, 0]


def kernel(x, edge_index, src_ids, tgt_ids, neg_ids, right, num_nodes, gt, cite_mask,
           edge_gnn_params, node_gnn_params, edge_pred_params, node_pred_params,
           fuse_src_W, fuse_src_b, fuse_tgt_W, fuse_tgt_b):
    n, d0 = x.shape
    h = edge_gnn_params[0][0].shape[1]
    src = edge_index[0]
    dst = edge_index[1]

    x_aug = jnp.zeros((_N_PAD, d0 + _CW), jnp.float32)
    x_aug = x_aug.at[:n, :d0].set(x)
    x_aug = x_aug.at[:, d0:].set(1.0)
    zeros_blk = jnp.zeros((_RPT, _CW), jnp.float32)

    def stack(pa, pb, i):
        return (jnp.stack([pa[i][0], pb[i][0]]),
                jnp.stack([pa[i][1], pb[i][1]]),
                jnp.stack([pa[i][2], pb[i][2]]))

    agg0 = _sc_seg_sum(x_aug, src, dst, zeros_blk)
    deg = agg0[:, d0:d0 + 1]
    wr, wn, b = stack(edge_gnn_params, node_gnn_params, 0)
    hc = _dual_layer(x_aug, agg0, deg, wr, wn, b, relu=True, split_input=False)

    for i in (1, 2):
        agg = _sc_seg_sum(hc, src, dst, zeros_blk)
        wr, wn, b = stack(edge_gnn_params, node_gnn_params, i)
        hc = _dual_layer(hc, agg, deg, wr, wn, b, relu=(i < 2), split_input=True)

    ids = jnp.concatenate([src_ids, tgt_ids, neg_ids])
    rows = _sc_gather(hc, ids)

    pos, neg, pred, loss = _heads(
        rows, gt, cite_mask,
        fuse_src_W, fuse_src_b, fuse_tgt_W, fuse_tgt_b,
        edge_pred_params, node_pred_params, h)

    loss = loss + jnp.asarray(right, loss.dtype) * 0.0 + jnp.asarray(num_nodes, loss.dtype) * 0.0
    return (loss, pos, neg, pred)
